# baseline (speedup 1.0000x reference)
import functools
import math

import jax
import jax.numpy as jnp
from jax import lax
from jax.experimental import pallas as pl
from jax.experimental.pallas import tpu as pltpu
from jax.experimental.pallas import tpu_sc as plsc

D_MODEL = 64
LANES = 16
NUM_WORKERS = 32
XCHUNK = 8
SCALE = math.sqrt(D_MODEL)


def _make_sc_lookup(n_x, k_x, d_model):
    assert d_model == D_MODEL
    assert n_x % (NUM_WORKERS * 2 * XCHUNK) == 0
    xrows_per_w = n_x // NUM_WORKERS
    n_chunks = xrows_per_w // XCHUNK
    chunk_rows = XCHUNK * k_x
    out_per_w = xrows_per_w * k_x

    mesh = plsc.VectorSubcoreMesh(core_axis_name="c", subcore_axis_name="s")

    @functools.partial(
        pl.kernel,
        mesh=mesh,
        out_type=jax.ShapeDtypeStruct((n_x * k_x, d_model), jnp.float32),
        compiler_params=pltpu.CompilerParams(use_tc_tiling_on_sc=False),
        scratch_types=[
            pltpu.VMEM((xrows_per_w, k_x), jnp.int32),
            pltpu.VMEM((chunk_rows, d_model), jnp.float32),
            pltpu.VMEM((chunk_rows, d_model), jnp.float32),
            pltpu.SemaphoreType.DMA,
            pltpu.SemaphoreType.DMA,
        ],
    )
    def sc_lookup(x_hbm, table_hbm, out_hbm, idx_v, rows0, rows1, sem0, sem1):
        wid = lax.axis_index("s") * 2 + lax.axis_index("c")
        xrow_base = wid * xrows_per_w
        out_base = wid * out_per_w

        rows = (rows0, rows1)
        sems = (sem0, sem1)

        pltpu.sync_copy(x_hbm.at[pl.ds(xrow_base, xrows_per_w)], idx_v)

        def fire(chunk, buf):
            for i in range(XCHUNK):
                pltpu.async_copy(
                    table_hbm.at[idx_v.at[chunk * XCHUNK + i]],
                    rows[buf].at[pl.ds(i * k_x, k_x)],
                    sems[buf],
                )

        def drain(buf):
            pltpu.make_async_copy(
                out_hbm.at[pl.ds(0, chunk_rows)], rows[buf], sems[buf]
            ).wait()

        fire(0, 0)
        fire(1, 1)

        def chunk_body(i, carry):
            for buf in range(2):
                c = 2 * i + buf
                drain(buf)

                def scale_body(g, acc):
                    for q in range(4):
                        for s in range(d_model // LANES):
                            sl = (4 * g + q, pl.ds(s * LANES, LANES))
                            rows[buf][sl] = rows[buf][sl] * SCALE
                    return acc

                lax.fori_loop(0, chunk_rows // 4, scale_body, 0)

                pltpu.sync_copy(
                    rows[buf],
                    out_hbm.at[pl.ds(out_base + c * chunk_rows, chunk_rows)],
                )

                @pl.when(c + 2 < n_chunks)
                def _():
                    fire(c + 2, buf)
            return carry

        lax.fori_loop(0, n_chunks // 2, chunk_body, 0)

    return sc_lookup


def kernel(x, table):
    n_x, k_x = x.shape
    out = _make_sc_lookup(n_x, k_x, table.shape[1])(x.astype(jnp.int32), table)
    return out.reshape(n_x, k_x, D_MODEL)


# speedup vs baseline: 1.0039x; 1.0039x over previous
import functools
import math

import jax
import jax.numpy as jnp
from jax import lax
from jax.experimental import pallas as pl
from jax.experimental.pallas import tpu as pltpu
from jax.experimental.pallas import tpu_sc as plsc

D_MODEL = 64
LANES = 16
NUM_WORKERS = 32
XCHUNK = 8
SCALE = math.sqrt(D_MODEL)


IDX_PAD = 128


def _make_sc_lookup(n_x, k_x, d_model):
    assert d_model == D_MODEL
    assert n_x % (NUM_WORKERS * 2 * XCHUNK) == 0
    xrows_per_w = n_x // NUM_WORKERS
    n_chunks = xrows_per_w // XCHUNK
    chunk_rows = XCHUNK * k_x
    out_per_w = xrows_per_w * k_x

    mesh = plsc.VectorSubcoreMesh(core_axis_name="c", subcore_axis_name="s")

    @functools.partial(
        pl.kernel,
        mesh=mesh,
        out_type=jax.ShapeDtypeStruct((n_x * k_x, d_model), jnp.float32),
        compiler_params=pltpu.CompilerParams(use_tc_tiling_on_sc=False),
        scratch_types=[
            pltpu.VMEM((xrows_per_w * IDX_PAD,), jnp.int32),
            pltpu.VMEM((chunk_rows, d_model), jnp.float32),
            pltpu.VMEM((chunk_rows, d_model), jnp.float32),
            pltpu.SemaphoreType.DMA,
            pltpu.SemaphoreType.DMA,
        ],
    )
    def sc_lookup(x_hbm, table_hbm, out_hbm, idx_v, rows0, rows1, sem0, sem1):
        wid = lax.axis_index("s") * 2 + lax.axis_index("c")
        out_base = wid * out_per_w

        rows = (rows0, rows1)
        sems = (sem0, sem1)

        pltpu.sync_copy(
            x_hbm.at[pl.ds(wid * xrows_per_w * IDX_PAD, xrows_per_w * IDX_PAD)],
            idx_v,
        )

        def fire(chunk, buf):
            for i in range(XCHUNK):
                pltpu.async_copy(
                    table_hbm.at[idx_v.at[pl.ds((chunk * XCHUNK + i) * IDX_PAD, k_x)]],
                    rows[buf].at[pl.ds(i * k_x, k_x)],
                    sems[buf],
                )

        def drain(buf):
            pltpu.make_async_copy(
                out_hbm.at[pl.ds(0, chunk_rows)], rows[buf], sems[buf]
            ).wait()

        fire(0, 0)
        fire(1, 1)

        def chunk_body(i, carry):
            for buf in range(2):
                c = 2 * i + buf
                drain(buf)

                def scale_body(g, acc):
                    for q in range(4):
                        for s in range(d_model // LANES):
                            sl = (4 * g + q, pl.ds(s * LANES, LANES))
                            rows[buf][sl] = rows[buf][sl] * SCALE
                    return acc

                lax.fori_loop(0, chunk_rows // 4, scale_body, 0)

                pltpu.sync_copy(
                    rows[buf],
                    out_hbm.at[pl.ds(out_base + c * chunk_rows, chunk_rows)],
                )

                @pl.when(c + 2 < n_chunks)
                def _():
                    fire(c + 2, buf)
            return carry

        lax.fori_loop(0, n_chunks // 2, chunk_body, 0)

    return sc_lookup


def kernel(x, table):
    n_x, k_x = x.shape
    xp = jnp.pad(x.astype(jnp.int32), ((0, 0), (0, IDX_PAD - k_x))).reshape(-1)
    out = _make_sc_lookup(n_x, k_x, table.shape[1])(xp, table)
    return out.reshape(n_x, k_x, D_MODEL)
